# baseline (speedup 1.0000x reference)
import functools

import jax
import jax.numpy as jnp
from jax import lax
from jax.experimental import pallas as pl
from jax.experimental.pallas import tpu as pltpu
from jax.experimental.pallas import tpu_sc as plsc

_L = 16
_NW = 32
_IDX_CHUNK = 128


def _make_sc_loss(B, D):
    b_per_w = B // _NW
    n_chunk = b_per_w // _IDX_CHUNK
    n_k = b_per_w // _L
    mesh = plsc.VectorSubcoreMesh(core_axis_name="c", subcore_axis_name="s")

    @functools.partial(
        pl.kernel,
        mesh=mesh,
        out_type=jax.ShapeDtypeStruct((_NW, _L), jnp.float32),
        scratch_types=[
            pltpu.VMEM((b_per_w,), jnp.int32),
            pltpu.VMEM((n_chunk, _IDX_CHUNK), jnp.int32),
            pltpu.VMEM((b_per_w,), jnp.float32),
            pltpu.VMEM((b_per_w, 2 * D), jnp.float32),
            pltpu.VMEM((b_per_w // 2, 2 * D), jnp.float32),
            pltpu.VMEM((_L,), jnp.float32),
            pltpu.SemaphoreType.DMA,
        ],
    )
    def sc_loss(tgt_hbm, emb_hbm, cent_hbm, out_hbm, idx_v, pidx_v, w1_v,
                prow_v, emb_v, acc_v, sem):
        wid = lax.axis_index("s") * 2 + lax.axis_index("c")
        base = wid * b_per_w
        pltpu.sync_copy(tgt_hbm.at[pl.ds(base, b_per_w)], idx_v)
        for c in range(n_k):
            t = idx_v[pl.ds(c * _L, _L)]
            pidx_v[c // 8, pl.ds((c % 8) * _L, _L)] = lax.shift_right_logical(
                t, 1)
            w1_v[pl.ds(c * _L, _L)] = lax.bitwise_and(t, 1).astype(jnp.float32)
        copies = [
            pltpu.async_copy(
                cent_hbm.at[pidx_v.at[g]],
                prow_v.at[pl.ds(g * _IDX_CHUNK, _IDX_CHUNK)],
                sem,
            )
            for g in range(n_chunk)
        ]
        pltpu.sync_copy(emb_hbm.at[pl.ds(wid * (b_per_w // 2), b_per_w // 2)],
                        emb_v)
        for cp in copies:
            cp.wait()

        zero = jnp.zeros((_L,), jnp.float32)
        vecs = D // _L

        def body(k, accs):
            w1ch = w1_v[pl.ds(k * _L, _L)]
            accs = list(accs)
            for r in range(_L):
                prow = k * _L + r
                erow = k * (_L // 2) + r // 2
                ebase = (r % 2) * D
                w1 = lax.gather(
                    w1ch,
                    jnp.full((_L, 1), r, jnp.int32),
                    lax.GatherDimensionNumbers(
                        offset_dims=(),
                        collapsed_slice_dims=(0,),
                        start_index_map=(0,),
                    ),
                    (1,),
                    mode=lax.GatherScatterMode.PROMISE_IN_BOUNDS,
                )
                w0 = 1.0 - w1
                for j in range(vecs):
                    e = emb_v[erow, pl.ds(ebase + j * _L, _L)]
                    c0 = prow_v[prow, pl.ds(j * _L, _L)]
                    c1 = prow_v[prow, pl.ds(D + j * _L, _L)]
                    d0 = e - c0
                    d1 = e - c1
                    accs[j] = accs[j] + (w0 * (d0 * d0) + w1 * (d1 * d1))
            return tuple(accs)

        accs = lax.fori_loop(0, n_k, body, (zero,) * vecs)
        total = accs[0]
        for j in range(1, vecs):
            total = total + accs[j]
        acc_v[...] = total
        pltpu.sync_copy(acc_v, out_hbm.at[wid])

    return sc_loss


def kernel(target, vector_embedding, centers):
    B, D = vector_embedding.shape
    tgt = target.astype(jnp.int32)
    cent2 = centers.reshape(-1, 2 * D)
    emb2 = vector_embedding.reshape(-1, 2 * D)
    partials = _make_sc_loss(B, D)(tgt, emb2, cent2)
    return jnp.sum(partials) * (0.5 / B)


# speedup vs baseline: 1.3478x; 1.3478x over previous
import functools

import jax
import jax.numpy as jnp
from jax import lax
from jax.experimental import pallas as pl
from jax.experimental.pallas import tpu as pltpu
from jax.experimental.pallas import tpu_sc as plsc

_L = 16
_NW = 32
_IDX_CHUNK = 128


def _make_sc_loss(B, D):
    b_per_w = B // _NW
    n_chunk = b_per_w // _IDX_CHUNK
    mesh = plsc.VectorSubcoreMesh(core_axis_name="c", subcore_axis_name="s")

    @functools.partial(
        pl.kernel,
        mesh=mesh,
        out_type=jax.ShapeDtypeStruct((_NW, _L), jnp.float32),
        scratch_types=[
            pltpu.VMEM((b_per_w,), jnp.int32),
            pltpu.VMEM((n_chunk, _IDX_CHUNK), jnp.int32),
            pltpu.VMEM((b_per_w, 2 * D), jnp.float32),
            pltpu.VMEM((b_per_w // 2, 2 * D), jnp.float32),
            pltpu.VMEM((_L,), jnp.float32),
            pltpu.SemaphoreType.DMA,
        ],
    )
    def sc_loss(tgt_hbm, emb_hbm, cent_hbm, out_hbm, idx_v, pidx_v, prow_v,
                emb_v, acc_v, sem):
        wid = lax.axis_index("s") * 2 + lax.axis_index("c")
        base = wid * b_per_w
        pltpu.sync_copy(tgt_hbm.at[pl.ds(base, b_per_w)], idx_v)
        for c in range(b_per_w // _L):
            pidx_v[c // 8, pl.ds((c % 8) * _L, _L)] = idx_v[pl.ds(c * _L, _L)]
        copies = [
            pltpu.async_copy(
                cent_hbm.at[pidx_v.at[g]],
                prow_v.at[pl.ds(g * _IDX_CHUNK, _IDX_CHUNK)],
                sem,
            )
            for g in range(n_chunk)
        ]
        pltpu.sync_copy(emb_hbm.at[pl.ds(wid * (b_per_w // 2), b_per_w // 2)],
                        emb_v)
        for cp in copies:
            cp.wait()

        zero = jnp.zeros((_L,), jnp.float32)
        vecs = D // _L

        def body(i2, accs):
            out = list(accs)
            for h in range(2):
                row = 2 * i2 + h
                for j in range(vecs):
                    e = emb_v[i2, pl.ds(h * D + j * _L, _L)]
                    c = prow_v[row, pl.ds(j * _L, _L)]
                    d = e - c
                    out[h * vecs + j] = out[h * vecs + j] + d * d
            return tuple(out)

        accs = lax.fori_loop(0, b_per_w // 2, body, (zero,) * (2 * vecs))
        total = accs[0]
        for j in range(1, 2 * vecs):
            total = total + accs[j]
        acc_v[...] = total
        pltpu.sync_copy(acc_v, out_hbm.at[wid])

    return sc_loss


def kernel(target, vector_embedding, centers):
    B, D = vector_embedding.shape
    tgt = target.astype(jnp.int32)
    cent_wide = jnp.pad(centers, ((0, 0), (0, D)))
    emb2 = vector_embedding.reshape(-1, 2 * D)
    partials = _make_sc_loss(B, D)(tgt, emb2, cent_wide)
    return jnp.sum(partials) * (0.5 / B)
